# speedup vs baseline: 22.2512x; 22.2512x over previous
import functools

import jax
import jax.numpy as jnp
from jax import lax
from jax.experimental import pallas as pl
from jax.experimental.pallas import tpu as pltpu
from jax.experimental.pallas import tpu_sc as plsc

_NC, _NS, _LANES = 2, 16, 16
_NW = _NC * _NS
_G = 8
_NBUF = 2


def _batcher_pairs(n2):
    pairs = []
    p = 1
    while p < n2:
        k = p
        while k >= 1:
            for j in range(k % p, n2 - k, 2 * k):
                for i in range(0, k):
                    if j + i + k < n2 and (i + j) // (p * 2) == (i + j + k) // (p * 2):
                        pairs.append((j + i, j + i + k))
            k //= 2
        p *= 2
    return pairs


def _median_net(n):
    assert n % 2 == 1
    n2 = 1
    while n2 < n:
        n2 *= 2
    pairs = [(i, j) for (i, j) in _batcher_pairs(n2) if i < n and j < n]
    out = n // 2
    needed = {out}
    ops = []
    for (i, j) in reversed(pairs):
        ni, nj = i in needed, j in needed
        if not (ni or nj):
            continue
        ops.append(("x" if (ni and nj) else ("n" if ni else "m"), i, j))
        needed.add(i)
        needed.add(j)
    ops.reverse()
    return ops


_MED17_OPS = _median_net(17)


def _apply_median_net(vals):
    w = list(vals)
    for kind, i, j in _MED17_OPS:
        a, b = w[i], w[j]
        if kind == "x":
            w[i] = jnp.minimum(a, b)
            w[j] = jnp.maximum(a, b)
        elif kind == "n":
            w[i] = jnp.minimum(a, b)
        else:
            w[j] = jnp.maximum(a, b)
    return w[len(vals) // 2]


def _mm_body(x_ref, w_ref, o_ref):
    o_ref[...] = jnp.dot(x_ref[...], w_ref[...],
                         preferred_element_type=jnp.float32)


def _matmul(x, w, blk=512):
    n, d = x.shape
    f = w.shape[1]
    return pl.pallas_call(
        _mm_body,
        grid=(n // blk,),
        in_specs=[
            pl.BlockSpec((blk, d), lambda i: (i, 0)),
            pl.BlockSpec((d, f), lambda i: (0, 0)),
        ],
        out_specs=pl.BlockSpec((blk, f), lambda i: (i, 0)),
        out_shape=jax.ShapeDtypeStruct((n, f), jnp.float32),
    )(x, w)


def _make_sc_median(np_, deg, f, relu):
    npw = np_ // _NW
    ngrp = npw // _G
    chunks = f // _LANES

    mesh = plsc.VectorSubcoreMesh(core_axis_name="c", subcore_axis_name="s",
                                  num_cores=_NC, num_subcores=_NS)

    @functools.partial(
        pl.kernel,
        mesh=mesh,
        out_type=jax.ShapeDtypeStruct((np_, f), jnp.float32),
        scratch_types=[
            pltpu.VMEM((npw * deg,), jnp.int32),
            pltpu.VMEM((_NBUF, _G * deg, f), jnp.float32),
            pltpu.VMEM((_NBUF, _G, f), jnp.float32),
            pltpu.VMEM((_NBUF, _G, f), jnp.float32),
            pltpu.SemaphoreType.DMA,
            pltpu.SemaphoreType.DMA,
            pltpu.SemaphoreType.DMA,
            pltpu.SemaphoreType.DMA,
            pltpu.SemaphoreType.DMA,
            pltpu.SemaphoreType.DMA,
        ],
    )
    def sc_median(table_hbm, adj_hbm, out_hbm, idx_v, rows_v, self_v, out_v,
                  g0, g1, s0, s1, o0, o1):
        gsem = (g0, g1)
        ssem = (s0, s1)
        osem = (o0, o1)
        c = lax.axis_index("c")
        s = lax.axis_index("s")
        wid = s * _NC + c
        base = wid * npw

        pltpu.sync_copy(adj_hbm.at[pl.ds(base * deg, npw * deg)], idx_v)

        def issue_gather(g, b):
            node0 = base + g * _G
            pltpu.async_copy(
                table_hbm.at[idx_v.at[pl.ds(g * (_G * deg), _G * deg)]],
                rows_v.at[b], gsem[b])
            pltpu.async_copy(table_hbm.at[pl.ds(node0, _G)],
                             self_v.at[b], ssem[b])

        def wait_gather(g, b):
            node0 = base + g * _G
            pltpu.make_async_copy(
                table_hbm.at[idx_v.at[pl.ds(g * (_G * deg), _G * deg)]],
                rows_v.at[b], gsem[b]).wait()
            pltpu.make_async_copy(table_hbm.at[pl.ds(node0, _G)],
                                  self_v.at[b], ssem[b]).wait()

        def issue_out(g, b):
            node0 = base + g * _G
            pltpu.async_copy(out_v.at[b],
                             out_hbm.at[pl.ds(node0, _G)], osem[b])

        def wait_out(g, b):
            node0 = base + g * _G
            pltpu.make_async_copy(out_v.at[b],
                                  out_hbm.at[pl.ds(node0, _G)],
                                  osem[b]).wait()

        def compute(b):
            def med_body(t, carry):
                i = t // chunks
                ch = (t % chunks) * _LANES
                vals = [self_v[b, i, pl.ds(ch, _LANES)]]
                for k in range(deg):
                    vals.append(rows_v[b, i * deg + k, pl.ds(ch, _LANES)])
                med = _apply_median_net(vals)
                if relu:
                    med = jnp.maximum(med, 0.0)
                out_v[b, i, pl.ds(ch, _LANES)] = med
                return carry

            lax.fori_loop(0, _G * chunks, med_body, 0)

        for b in range(_NBUF):
            issue_gather(jnp.int32(b), b)

        def outer(step, carry):
            for b in range(_NBUF):
                g = step * _NBUF + b
                wait_gather(g, b)

                @pl.when(g >= _NBUF)
                def _():
                    wait_out(g - _NBUF, b)

                compute(b)
                issue_out(g, b)

                @pl.when(g + _NBUF < ngrp)
                def _():
                    issue_gather(g + _NBUF, b)

            return carry

        lax.fori_loop(0, ngrp // _NBUF, outer, 0)

        for b in range(_NBUF):
            wait_out(jnp.int32(ngrp - _NBUF + b), b)

    return sc_median


@jax.jit
def kernel(x, adj, W1, W2):
    n, d = x.shape
    deg = adj.shape[1]
    f1 = W1.shape[1]
    f2 = W2.shape[1]

    align = _NW * _G
    np_ = ((n + align - 1) // align) * align

    x_pad = jnp.pad(x, ((0, np_ - n), (0, 0)))
    adj_flat = jnp.pad(adj, ((0, np_ - n), (0, 0))).reshape(-1)

    sc_relu = _make_sc_median(np_, deg, f1, relu=True)
    sc_plain = _make_sc_median(np_, deg, f2, relu=False)

    h = _matmul(x_pad, W1)
    h = sc_relu(h, adj_flat)
    h = _matmul(h, W2)
    out = sc_plain(h, adj_flat)
    return out[:n]


# speedup vs baseline: 22.4740x; 1.0100x over previous
import functools

import jax
import jax.numpy as jnp
from jax import lax
from jax.experimental import pallas as pl
from jax.experimental.pallas import tpu as pltpu
from jax.experimental.pallas import tpu_sc as plsc

_NC, _NS, _LANES = 2, 16, 16
_NW = _NC * _NS
_G = 8
_NBUF = 3
_NSPLIT = 2


def _batcher_pairs(n2):
    pairs = []
    p = 1
    while p < n2:
        k = p
        while k >= 1:
            for j in range(k % p, n2 - k, 2 * k):
                for i in range(0, k):
                    if j + i + k < n2 and (i + j) // (p * 2) == (i + j + k) // (p * 2):
                        pairs.append((j + i, j + i + k))
            k //= 2
        p *= 2
    return pairs


def _select_net(n, outs):
    pairs = _batcher_pairs(n)
    needed = set(outs)
    ops = []
    for (i, j) in reversed(pairs):
        ni, nj = i in needed, j in needed
        if not (ni or nj):
            continue
        ops.append(("x" if (ni and nj) else ("n" if ni else "m"), i, j))
        needed.add(i)
        needed.add(j)
    ops.reverse()
    return ops


_SEL16_OPS = _select_net(16, {7, 8})


def _median17(self_val, nbrs):
    w = list(nbrs)
    for kind, i, j in _SEL16_OPS:
        a, b = w[i], w[j]
        if kind == "x":
            w[i] = jnp.minimum(a, b)
            w[j] = jnp.maximum(a, b)
        elif kind == "n":
            w[i] = jnp.minimum(a, b)
        else:
            w[j] = jnp.maximum(a, b)
    return jnp.maximum(w[7], jnp.minimum(self_val, w[8]))


def _mm_body(x_ref, w_ref, o_ref):
    o_ref[...] = jnp.dot(x_ref[...], w_ref[...],
                         preferred_element_type=jnp.float32)


def _matmul(x, w, blk=512):
    n, d = x.shape
    f = w.shape[1]
    return pl.pallas_call(
        _mm_body,
        grid=(n // blk,),
        in_specs=[
            pl.BlockSpec((blk, d), lambda i: (i, 0)),
            pl.BlockSpec((d, f), lambda i: (0, 0)),
        ],
        out_specs=pl.BlockSpec((blk, f), lambda i: (i, 0)),
        out_shape=jax.ShapeDtypeStruct((n, f), jnp.float32),
    )(x, w)


def _make_sc_median(np_, deg, f, relu):
    npw = np_ // _NW
    ngrp = npw // _G
    chunks = f // _LANES

    mesh = plsc.VectorSubcoreMesh(core_axis_name="c", subcore_axis_name="s",
                                  num_cores=_NC, num_subcores=_NS)

    @functools.partial(
        pl.kernel,
        mesh=mesh,
        out_type=jax.ShapeDtypeStruct((np_, f), jnp.float32),
        scratch_types=[
            pltpu.VMEM((npw * deg,), jnp.int32),
            pltpu.VMEM((_NBUF, _G * deg, f), jnp.float32),
            pltpu.VMEM((_NBUF, _G, f), jnp.float32),
            pltpu.VMEM((_NBUF, _G, f), jnp.float32),
        ] + [pltpu.SemaphoreType.DMA] * (3 * _NBUF),
    )
    def sc_median(table_hbm, adj_hbm, out_hbm, idx_v, rows_v, self_v, out_v,
                  *sems):
        gsem = sems[0:_NBUF]
        ssem = sems[_NBUF:2 * _NBUF]
        osem = sems[2 * _NBUF:3 * _NBUF]
        c = lax.axis_index("c")
        s = lax.axis_index("s")
        wid = s * _NC + c
        base = wid * npw

        pltpu.sync_copy(adj_hbm.at[pl.ds(base * deg, npw * deg)], idx_v)

        part = (_G * deg) // _NSPLIT

        def issue_gather(g, b):
            node0 = base + g * _G
            for p in range(_NSPLIT):
                pltpu.async_copy(
                    table_hbm.at[idx_v.at[pl.ds(g * (_G * deg) + p * part,
                                                part)]],
                    rows_v.at[b].at[pl.ds(p * part, part)], gsem[b])
            pltpu.async_copy(table_hbm.at[pl.ds(node0, _G)],
                             self_v.at[b], ssem[b])

        def wait_gather(g, b):
            node0 = base + g * _G
            for p in range(_NSPLIT):
                pltpu.make_async_copy(
                    table_hbm.at[idx_v.at[pl.ds(g * (_G * deg) + p * part,
                                                part)]],
                    rows_v.at[b].at[pl.ds(p * part, part)], gsem[b]).wait()
            pltpu.make_async_copy(table_hbm.at[pl.ds(node0, _G)],
                                  self_v.at[b], ssem[b]).wait()

        def issue_out(g, b):
            node0 = base + g * _G
            pltpu.async_copy(out_v.at[b],
                             out_hbm.at[pl.ds(node0, _G)], osem[b])

        def wait_out(g, b):
            node0 = base + g * _G
            pltpu.make_async_copy(out_v.at[b],
                                  out_hbm.at[pl.ds(node0, _G)],
                                  osem[b]).wait()

        def compute(b):
            half = chunks // 2

            def med_body(t, carry):
                i = t // half
                ch = (t % half) * (2 * _LANES)
                for c in (ch, ch + _LANES):
                    sv = self_v[b, i, pl.ds(c, _LANES)]
                    nbrs = [rows_v[b, i * deg + k, pl.ds(c, _LANES)]
                            for k in range(deg)]
                    med = _median17(sv, nbrs)
                    if relu:
                        med = jnp.maximum(med, 0.0)
                    out_v[b, i, pl.ds(c, _LANES)] = med
                return carry

            lax.fori_loop(0, _G * half, med_body, 0)

        for b in range(_NBUF):
            issue_gather(jnp.int32(b), b)

        def outer(step, carry):
            for b in range(_NBUF):
                g = step * _NBUF + b
                wait_gather(g, b)

                @pl.when(g >= _NBUF)
                def _():
                    wait_out(g - _NBUF, b)

                compute(b)
                issue_out(g, b)

                @pl.when(g + _NBUF < ngrp)
                def _():
                    issue_gather(g + _NBUF, b)

            return carry

        lax.fori_loop(0, ngrp // _NBUF, outer, 0)

        for g in range((ngrp // _NBUF) * _NBUF, ngrp):
            b = g % _NBUF
            wait_gather(jnp.int32(g), b)
            wait_out(jnp.int32(g - _NBUF), b)
            compute(b)
            issue_out(jnp.int32(g), b)

        for g in range(ngrp - _NBUF, ngrp):
            wait_out(jnp.int32(g), g % _NBUF)

    return sc_median


@jax.jit
def kernel(x, adj, W1, W2):
    n, d = x.shape
    deg = adj.shape[1]
    f1 = W1.shape[1]
    f2 = W2.shape[1]

    align = _NW * _G
    np_ = ((n + align - 1) // align) * align

    x_pad = jnp.pad(x, ((0, np_ - n), (0, 0)))
    adj_flat = jnp.pad(adj, ((0, np_ - n), (0, 0))).reshape(-1)

    sc_relu = _make_sc_median(np_, deg, f1, relu=True)
    sc_plain = _make_sc_median(np_, deg, f2, relu=False)

    h = _matmul(x_pad, W1)
    h = sc_relu(h, adj_flat)
    h = _matmul(h, W2)
    out = sc_plain(h, adj_flat)
    return out[:n]
